# Initial kernel scaffold; baseline (speedup 1.0000x reference)
import jax
import jax.numpy as jnp
from jax.experimental import pallas as pl


def kernel(x, weights):
    raise NotImplementedError("write your pallas kernel here")


# speedup vs baseline: 2.9627x; 2.9627x over previous
import functools

import jax
import jax.numpy as jnp
from jax import lax
from jax.experimental import pallas as pl
from jax.experimental.pallas import tpu as pltpu
from jax.experimental.pallas import tpu_sc as plsc

_D = 128
_C = 128


@functools.partial(jax.jit, static_argnums=(2,))
def _sc_gather(weights, idx, n):
    info = plsc.get_sparse_core_info()
    nw = info.num_cores * info.num_subcores
    n_chunks = n // (nw * _C)
    b_per_w = n // nw

    idx3 = idx.reshape(nw, n_chunks, _C)
    mesh = plsc.VectorSubcoreMesh(core_axis_name="c", subcore_axis_name="s")

    @functools.partial(
        pl.kernel,
        mesh=mesh,
        out_type=jax.ShapeDtypeStruct((n, _D), jnp.float32),
        scratch_types=[
            pltpu.VMEM((n_chunks, _C), jnp.int32),
            pltpu.VMEM((_C, _D), jnp.float32),
            pltpu.SemaphoreType.DMA,
        ],
    )
    def gather(table_hbm, idx_hbm, out_hbm, idx_v, rows_v, gsem):
        wid = lax.axis_index("s") * info.num_cores + lax.axis_index("c")
        base = wid * b_per_w
        pltpu.sync_copy(idx_hbm.at[wid], idx_v)

        def body(c, carry):
            pltpu.async_copy(table_hbm.at[idx_v.at[c]], rows_v, gsem).wait()
            pltpu.sync_copy(rows_v, out_hbm.at[pl.ds(base + c * _C, _C)])
            return carry

        lax.fori_loop(0, n_chunks, body, 0, unroll=False)

    return gather(weights, idx3)


def kernel(x, weights):
    b, s = x.shape
    n = b * s
    idx = x.reshape(n).astype(jnp.int32)
    out = _sc_gather(weights, idx, n)
    return out.reshape(b, s, _D)


# speedup vs baseline: 3.3425x; 1.1282x over previous
import functools

import jax
import jax.numpy as jnp
from jax import lax
from jax.experimental import pallas as pl
from jax.experimental.pallas import tpu as pltpu
from jax.experimental.pallas import tpu_sc as plsc

_D = 128
_C = 128


@functools.partial(jax.jit, static_argnums=(2,))
def _sc_gather(weights, idx, n):
    info = plsc.get_sparse_core_info()
    nw = info.num_cores * info.num_subcores
    n_chunks = n // (nw * _C)
    b_per_w = n // nw
    assert n_chunks >= 3 and n_chunks % 2 == 0

    idx3 = idx.reshape(nw, n_chunks, _C)
    mesh = plsc.VectorSubcoreMesh(core_axis_name="c", subcore_axis_name="s")

    @functools.partial(
        pl.kernel,
        mesh=mesh,
        out_type=jax.ShapeDtypeStruct((n, _D), jnp.float32),
        scratch_types=[
            pltpu.VMEM((n_chunks, _C), jnp.int32),
            pltpu.VMEM((2, _C, _D), jnp.float32),
            pltpu.SemaphoreType.DMA,
            pltpu.SemaphoreType.DMA,
            pltpu.SemaphoreType.DMA,
            pltpu.SemaphoreType.DMA,
        ],
    )
    def gather(table_hbm, idx_hbm, out_hbm, idx_v, rows_v, gs0, gs1, os0, os1):
        gs = (gs0, gs1)
        osm = (os0, os1)
        wid = lax.axis_index("s") * info.num_cores + lax.axis_index("c")
        base = wid * b_per_w
        pltpu.sync_copy(idx_hbm.at[wid], idx_v)

        def g_start(cc, b):
            pltpu.async_copy(table_hbm.at[idx_v.at[cc]], rows_v.at[b], gs[b])

        def g_wait(cc, b):
            pltpu.make_async_copy(
                table_hbm.at[idx_v.at[cc]], rows_v.at[b], gs[b]).wait()

        def o_start(cc, b):
            pltpu.async_copy(
                rows_v.at[b], out_hbm.at[pl.ds(base + cc * _C, _C)], osm[b])

        def o_wait(cc, b):
            pltpu.make_async_copy(
                rows_v.at[b], out_hbm.at[pl.ds(base + cc * _C, _C)], osm[b]).wait()

        g_start(0, 0)
        g_start(1, 1)
        g_wait(0, 0)
        o_start(0, 0)

        def body(o, carry):
            c = 1 + 2 * o
            for db in range(2):
                cc = c + db
                b = 1 - db
                nb = db
                o_wait(cc - 1, nb)
                g_start(cc + 1, nb)
                g_wait(cc, b)
                o_start(cc, b)
            return carry

        lax.fori_loop(0, (n_chunks - 2) // 2, body, 0, unroll=False)

        last = n_chunks - 1
        g_wait(last, 1)
        o_start(last, 1)
        o_wait(last - 1, 0)
        o_wait(last, 1)

    return gather(weights, idx3)


def kernel(x, weights):
    b, s = x.shape
    n = b * s
    idx = x.reshape(n).astype(jnp.int32)
    out = _sc_gather(weights, idx, n)
    return out.reshape(b, s, _D)
